# Initial kernel scaffold; baseline (speedup 1.0000x reference)
import jax
import jax.numpy as jnp
from jax.experimental import pallas as pl


def kernel(x, embed):
    raise NotImplementedError("write your pallas kernel here")


# speedup vs baseline: 4.0374x; 4.0374x over previous
import functools

import jax
import jax.numpy as jnp
from jax import lax
from jax.experimental import pallas as pl
from jax.experimental.pallas import tpu as pltpu
from jax.experimental.pallas import tpu_sc as plsc

_BT = 2048
_KC = 2048


def _argmin_body(x_ref, e_ref, idx_ref):
    n_codes = e_ref.shape[0]
    bt = x_ref.shape[0]
    x = x_ref[...]
    x_norm = jnp.sum(x * x, axis=1, keepdims=True)
    best = jnp.full((bt,), jnp.inf, jnp.float32)
    best_i = jnp.zeros((bt,), jnp.int32)
    for c in range(n_codes // _KC):
        e = e_ref[pl.ds(c * _KC, _KC), :]
        e_norm = jnp.sum(e * e, axis=1)
        xe = lax.dot_general(x, e, (((1,), (1,)), ((), ())),
                             preferred_element_type=jnp.float32)
        d = x_norm + e_norm[None, :] - 2.0 * xe
        m = jnp.min(d, axis=1)
        iota = lax.broadcasted_iota(jnp.int32, (bt, _KC), 1)
        am = jnp.min(jnp.where(d == m[:, None], iota, n_codes), axis=1)
        upd = m < best
        best = jnp.where(upd, m, best)
        best_i = jnp.where(upd, am + c * _KC, best_i)
    idx_ref[...] = best_i


def _nearest_indices(flat_x, embed):
    tokens, dim = flat_x.shape
    n_codes = embed.shape[0]
    return pl.pallas_call(
        _argmin_body,
        grid=(tokens // _BT,),
        in_specs=[
            pl.BlockSpec((_BT, dim), lambda i: (i, 0)),
            pl.BlockSpec((n_codes, dim), lambda i: (0, 0)),
        ],
        out_specs=pl.BlockSpec((_BT,), lambda i: (i,)),
        out_shape=jax.ShapeDtypeStruct((tokens,), jnp.int32),
    )(flat_x, embed)


@functools.cache
def _make_sc_gather(tokens, dim):
    info = plsc.get_sparse_core_info()
    nw = info.num_cores * info.num_subcores
    b_per_w = tokens // nw
    mesh = plsc.VectorSubcoreMesh(core_axis_name="c", subcore_axis_name="s")

    @functools.partial(
        pl.kernel, mesh=mesh,
        compiler_params=pltpu.CompilerParams(use_tc_tiling_on_sc=False),
        out_type=jax.ShapeDtypeStruct((tokens, dim), jnp.float32),
        scratch_types=[
            pltpu.VMEM((b_per_w,), jnp.int32),
            pltpu.VMEM((b_per_w, dim), jnp.float32),
            pltpu.SemaphoreType.DMA,
        ],
    )
    def gather(table_hbm, idx_hbm, out_hbm, idx_v, rows_v, sem):
        wid = lax.axis_index("s") * info.num_cores + lax.axis_index("c")
        base = wid * b_per_w
        pltpu.sync_copy(idx_hbm.at[pl.ds(base, b_per_w)], idx_v)
        pltpu.async_copy(table_hbm.at[idx_v], rows_v, sem).wait()
        pltpu.sync_copy(rows_v, out_hbm.at[pl.ds(base, b_per_w)])

    return gather


def kernel(x, embed):
    tokens = x.shape[0] * x.shape[1]
    dim = x.shape[2]
    flat_x = x.reshape(tokens, dim)
    idx = _nearest_indices(flat_x, embed)
    quantized = _make_sc_gather(tokens, dim)(embed, idx)
    return quantized.reshape(x.shape)


# speedup vs baseline: 4.0413x; 1.0010x over previous
import functools

import jax
import jax.numpy as jnp
from jax import lax
from jax.experimental import pallas as pl
from jax.experimental.pallas import tpu as pltpu
from jax.experimental.pallas import tpu_sc as plsc

_BT = 2048
_KC = 2048


def _argmin_body(x_ref, e_ref, idx_ref):
    n_codes = e_ref.shape[0]
    bt = x_ref.shape[0]
    x = x_ref[...]
    x_norm = jnp.sum(x * x, axis=1, keepdims=True)
    best = jnp.full((bt,), jnp.inf, jnp.float32)
    best_i = jnp.zeros((bt,), jnp.int32)
    for c in range(n_codes // _KC):
        e = e_ref[pl.ds(c * _KC, _KC), :]
        e_norm = jnp.sum(e * e, axis=1)
        xe = lax.dot_general(x, e, (((1,), (1,)), ((), ())),
                             preferred_element_type=jnp.float32)
        d = x_norm + e_norm[None, :] - 2.0 * xe
        m = jnp.min(d, axis=1)
        iota = lax.broadcasted_iota(jnp.int32, (bt, _KC), 1)
        am = jnp.min(jnp.where(d == m[:, None], iota, n_codes), axis=1)
        upd = m < best
        best = jnp.where(upd, m, best)
        best_i = jnp.where(upd, am + c * _KC, best_i)
    idx_ref[...] = best_i


def _nearest_indices(flat_x, embed):
    tokens, dim = flat_x.shape
    n_codes = embed.shape[0]
    return pl.pallas_call(
        _argmin_body,
        grid=(tokens // _BT,),
        compiler_params=pltpu.CompilerParams(
            dimension_semantics=("parallel",)),
        in_specs=[
            pl.BlockSpec((_BT, dim), lambda i: (i, 0)),
            pl.BlockSpec((n_codes, dim), lambda i: (0, 0)),
        ],
        out_specs=pl.BlockSpec((_BT,), lambda i: (i,)),
        out_shape=jax.ShapeDtypeStruct((tokens,), jnp.int32),
    )(flat_x, embed)


@functools.cache
def _make_sc_gather(tokens, dim):
    info = plsc.get_sparse_core_info()
    nw = info.num_cores * info.num_subcores
    b_per_w = tokens // nw
    mesh = plsc.VectorSubcoreMesh(core_axis_name="c", subcore_axis_name="s")

    @functools.partial(
        pl.kernel, mesh=mesh,
        compiler_params=pltpu.CompilerParams(use_tc_tiling_on_sc=False),
        out_type=jax.ShapeDtypeStruct((tokens, dim), jnp.float32),
        scratch_types=[
            pltpu.VMEM((b_per_w,), jnp.int32),
            pltpu.VMEM((b_per_w, dim), jnp.float32),
            pltpu.SemaphoreType.DMA,
        ],
    )
    def gather(table_hbm, idx_hbm, out_hbm, idx_v, rows_v, sem):
        wid = lax.axis_index("s") * info.num_cores + lax.axis_index("c")
        base = wid * b_per_w
        pltpu.sync_copy(idx_hbm.at[pl.ds(base, b_per_w)], idx_v)
        pltpu.async_copy(table_hbm.at[idx_v], rows_v, sem).wait()
        pltpu.sync_copy(rows_v, out_hbm.at[pl.ds(base, b_per_w)])

    return gather


def kernel(x, embed):
    tokens = x.shape[0] * x.shape[1]
    dim = x.shape[2]
    flat_x = x.reshape(tokens, dim)
    idx = _nearest_indices(flat_x, embed)
    quantized = _make_sc_gather(tokens, dim)(embed, idx)
    return quantized.reshape(x.shape)


# speedup vs baseline: 4.5266x; 1.1201x over previous
import functools

import jax
import jax.numpy as jnp
from jax import lax
from jax.experimental import pallas as pl
from jax.experimental.pallas import tpu as pltpu
from jax.experimental.pallas import tpu_sc as plsc

_BT = 2048
_KC = 2048


def _argmin_body(x_ref, e_ref, idx_ref):
    n_codes = e_ref.shape[0]
    bt = x_ref.shape[0]
    x = x_ref[...]
    x_norm = jnp.sum(x * x, axis=1, keepdims=True)
    lane = lax.broadcasted_iota(jnp.int32, (bt, 128), 1)
    m = jnp.full((bt, 128), jnp.inf, jnp.float32)
    mi = jnp.zeros((bt, 128), jnp.int32)
    for c in range(n_codes // _KC):
        e = e_ref[pl.ds(c * _KC, _KC), :]
        e_norm = jnp.sum(e * e, axis=1)
        xe = lax.dot_general(x, e, (((1,), (1,)), ((), ())),
                             preferred_element_type=jnp.float32)
        d = x_norm + e_norm[None, :] - 2.0 * xe
        for g in range(_KC // 128):
            dg = d[:, g * 128:(g + 1) * 128]
            upd = dg < m
            m = jnp.where(upd, dg, m)
            mi = jnp.where(upd, lane + (c * _KC + g * 128), mi)
    gm = jnp.min(m, axis=1, keepdims=True)
    idx_ref[...] = jnp.min(jnp.where(m == gm, mi, n_codes), axis=1)


def _nearest_indices(flat_x, embed):
    tokens, dim = flat_x.shape
    n_codes = embed.shape[0]
    return pl.pallas_call(
        _argmin_body,
        grid=(tokens // _BT,),
        compiler_params=pltpu.CompilerParams(
            dimension_semantics=("parallel",)),
        in_specs=[
            pl.BlockSpec((_BT, dim), lambda i: (i, 0)),
            pl.BlockSpec((n_codes, dim), lambda i: (0, 0)),
        ],
        out_specs=pl.BlockSpec((_BT,), lambda i: (i,)),
        out_shape=jax.ShapeDtypeStruct((tokens,), jnp.int32),
    )(flat_x, embed)


@functools.cache
def _make_sc_gather(tokens, dim):
    info = plsc.get_sparse_core_info()
    nw = info.num_cores * info.num_subcores
    b_per_w = tokens // nw
    mesh = plsc.VectorSubcoreMesh(core_axis_name="c", subcore_axis_name="s")

    @functools.partial(
        pl.kernel, mesh=mesh,
        compiler_params=pltpu.CompilerParams(use_tc_tiling_on_sc=False),
        out_type=jax.ShapeDtypeStruct((tokens, dim), jnp.float32),
        scratch_types=[
            pltpu.VMEM((b_per_w,), jnp.int32),
            pltpu.VMEM((b_per_w, dim), jnp.float32),
            pltpu.SemaphoreType.DMA,
        ],
    )
    def gather(table_hbm, idx_hbm, out_hbm, idx_v, rows_v, sem):
        wid = lax.axis_index("s") * info.num_cores + lax.axis_index("c")
        base = wid * b_per_w
        pltpu.sync_copy(idx_hbm.at[pl.ds(base, b_per_w)], idx_v)
        pltpu.async_copy(table_hbm.at[idx_v], rows_v, sem).wait()
        pltpu.sync_copy(rows_v, out_hbm.at[pl.ds(base, b_per_w)])

    return gather


def kernel(x, embed):
    tokens = x.shape[0] * x.shape[1]
    dim = x.shape[2]
    flat_x = x.reshape(tokens, dim)
    idx = _nearest_indices(flat_x, embed)
    quantized = _make_sc_gather(tokens, dim)(embed, idx)
    return quantized.reshape(x.shape)


# speedup vs baseline: 4.7494x; 1.0492x over previous
import functools

import jax
import jax.numpy as jnp
from jax import lax
from jax.experimental import pallas as pl
from jax.experimental.pallas import tpu as pltpu
from jax.experimental.pallas import tpu_sc as plsc

_BT = 2048
_KC = 2048


def _argmin_body(x_ref, e_ref, idx_ref):
    n_codes = e_ref.shape[0]
    bt = x_ref.shape[0]
    x = x_ref[...]
    x_norm = jnp.sum(x * x, axis=1, keepdims=True)
    xm2 = x * (-2.0)
    lane = lax.broadcasted_iota(jnp.int32, (bt, 128), 1)
    m = jnp.full((bt, 128), jnp.inf, jnp.float32)
    mi = jnp.zeros((bt, 128), jnp.int32)
    for c in range(n_codes // _KC):
        e = e_ref[pl.ds(c * _KC, _KC), :]
        e_norm = jnp.sum(e * e, axis=1)
        xe2 = lax.dot_general(xm2, e, (((1,), (1,)), ((), ())),
                              preferred_element_type=jnp.float32)
        d = (x_norm + e_norm[None, :]) + xe2
        for g in range(_KC // 128):
            dg = d[:, g * 128:(g + 1) * 128]
            upd = dg < m
            m = jnp.where(upd, dg, m)
            mi = jnp.where(upd, lane + (c * _KC + g * 128), mi)
    gm = jnp.min(m, axis=1, keepdims=True)
    idx_ref[...] = jnp.min(jnp.where(m == gm, mi, n_codes), axis=1)


def _nearest_indices(flat_x, embed):
    tokens, dim = flat_x.shape
    n_codes = embed.shape[0]
    return pl.pallas_call(
        _argmin_body,
        grid=(tokens // _BT,),
        compiler_params=pltpu.CompilerParams(
            dimension_semantics=("parallel",)),
        in_specs=[
            pl.BlockSpec((_BT, dim), lambda i: (i, 0)),
            pl.BlockSpec((n_codes, dim), lambda i: (0, 0)),
        ],
        out_specs=pl.BlockSpec((_BT,), lambda i: (i,)),
        out_shape=jax.ShapeDtypeStruct((tokens,), jnp.int32),
    )(flat_x, embed)


@functools.cache
def _make_sc_gather(tokens, dim):
    info = plsc.get_sparse_core_info()
    nw = info.num_cores * info.num_subcores
    b_per_w = tokens // nw
    mesh = plsc.VectorSubcoreMesh(core_axis_name="c", subcore_axis_name="s")

    @functools.partial(
        pl.kernel, mesh=mesh,
        compiler_params=pltpu.CompilerParams(use_tc_tiling_on_sc=False),
        out_type=jax.ShapeDtypeStruct((tokens, dim), jnp.float32),
        scratch_types=[
            pltpu.VMEM((b_per_w,), jnp.int32),
            pltpu.VMEM((b_per_w, dim), jnp.float32),
            pltpu.SemaphoreType.DMA,
        ],
    )
    def gather(table_hbm, idx_hbm, out_hbm, idx_v, rows_v, sem):
        wid = lax.axis_index("s") * info.num_cores + lax.axis_index("c")
        base = wid * b_per_w
        pltpu.sync_copy(idx_hbm.at[pl.ds(base, b_per_w)], idx_v)
        pltpu.async_copy(table_hbm.at[idx_v], rows_v, sem).wait()
        pltpu.sync_copy(rows_v, out_hbm.at[pl.ds(base, b_per_w)])

    return gather


def kernel(x, embed):
    tokens = x.shape[0] * x.shape[1]
    dim = x.shape[2]
    flat_x = x.reshape(tokens, dim)
    idx = _nearest_indices(flat_x, embed)
    quantized = _make_sc_gather(tokens, dim)(embed, idx)
    return quantized.reshape(x.shape)
